# baseline (speedup 1.0000x reference)
import functools

import jax
import jax.numpy as jnp
from jax import lax
from jax.experimental import pallas as pl
from jax.experimental.pallas import tpu as pltpu
from jax.experimental.pallas import tpu_sc as plsc

_NC = 2
_NS = 16
_NW = _NC * _NS
_LANES = 16
_EPS = 1e-5


def _proj_body(x_ref, ws_ref, wd_ref, p_ref, d_ref):
    xv = x_ref[...]
    p_ref[...] = jnp.dot(
        xv, ws_ref[...], preferred_element_type=jnp.float32
    ).astype(jnp.bfloat16)
    d_ref[...] = jnp.dot(
        xv, wd_ref[...], preferred_element_type=jnp.float32
    ).astype(jnp.bfloat16)


def _stats_body(ea_ref, gp_ref, gd_ref, we_ref, sum_ref, sq_ref):
    y = (gp_ref[...].astype(jnp.float32) + gd_ref[...].astype(jnp.float32)
         + jnp.dot(ea_ref[...], we_ref[...],
                   preferred_element_type=jnp.float32))
    s = jnp.sum(y, axis=0, keepdims=True)
    q = jnp.sum(y * y, axis=0, keepdims=True)

    @pl.when(pl.program_id(0) == 0)
    def _():
        sum_ref[...] = s
        sq_ref[...] = q

    @pl.when(pl.program_id(0) > 0)
    def _():
        sum_ref[...] += s
        sq_ref[...] += q


def _act_body(n_edges, d_feat, ea_ref, gp_ref, gd_ref, we_ref, sum_ref,
              sq_ref, gam_ref, bet_ref, m_ref):
    inv_e = 1.0 / n_edges
    mean = sum_ref[...] * inv_e
    var = sq_ref[...] * inv_e - mean * mean
    inv = lax.rsqrt(var + _EPS)
    scale = gam_ref[...] * inv
    shift = bet_ref[...] - mean * scale
    y = (gp_ref[...].astype(jnp.float32) + gd_ref[...].astype(jnp.float32)
         + jnp.dot(ea_ref[...], we_ref[...],
                   preferred_element_type=jnp.float32))
    z = y * scale + shift
    zm = z[:, :d_feat]
    zg = z[:, d_feat:]
    m_ref[...] = jax.nn.sigmoid(zm) * jax.nn.softplus(zg)


def _final_body(n_nodes, part_ref, x_ref, gn_ref, bn_ref, out_ref):
    agg = part_ref[0, :n_nodes, :] + part_ref[1, :n_nodes, :]
    mean = jnp.mean(agg, axis=0, keepdims=True)
    cent = agg - mean
    var = jnp.mean(cent * cent, axis=0, keepdims=True)
    z = cent * lax.rsqrt(var + _EPS) * gn_ref[...] + bn_ref[...]
    out_ref[...] = jax.nn.sigmoid(z + x_ref[...])


def kernel(x, edge_index, edge_attr, W_mlpt, b_mlpt, gamma_mlpt, beta_mlpt,
           W_gate, b_gate, gamma_gate, beta_gate, gamma_node, beta_node):
    del b_mlpt, b_gate
    n_nodes, d = x.shape
    n_edges = edge_attr.shape[0]
    d2 = 2 * d

    w_src = jnp.concatenate([W_mlpt[:d], W_gate[:d]], axis=1)
    w_dst = jnp.concatenate([W_mlpt[d:2 * d], W_gate[d:2 * d]], axis=1)
    w_edge = jnp.concatenate([W_mlpt[2 * d:], W_gate[2 * d:]], axis=1)
    gam = jnp.concatenate([gamma_mlpt, gamma_gate]).reshape(1, d2)
    bet = jnp.concatenate([beta_mlpt, beta_gate]).reshape(1, d2)
    src = edge_index[0]
    dst = edge_index[1]

    p_tab, d_tab = pl.pallas_call(
        _proj_body,
        out_shape=[jax.ShapeDtypeStruct((n_nodes, d2), jnp.bfloat16),
                   jax.ShapeDtypeStruct((n_nodes, d2), jnp.bfloat16)],
    )(x, w_src, w_dst)

    ew = n_edges // _NW
    blk = 80
    n_chunks = ew // blk
    mesh = plsc.VectorSubcoreMesh(core_axis_name="c", subcore_axis_name="s",
                                  num_cores=_NC, num_subcores=_NS)

    dw = d2 // 2

    @functools.partial(
        pl.kernel,
        out_type=[jax.ShapeDtypeStruct((n_edges, dw), jnp.int32),
                  jax.ShapeDtypeStruct((n_edges, dw), jnp.int32)],
        mesh=mesh,
        scratch_types=[
            pltpu.VMEM((blk,), jnp.int32),
            pltpu.VMEM((blk,), jnp.int32),
            pltpu.VMEM((blk, dw), jnp.int32),
            pltpu.VMEM((blk, dw), jnp.int32),
            pltpu.SemaphoreType.DMA,
            pltpu.SemaphoreType.DMA,
        ],
    )
    def _gather_rows(p_hbm, dn_hbm, src_hbm, dst_hbm, gp_hbm, gd_hbm,
                     sidx, didx, prow, drow, sem_p, sem_d):
        wid = lax.axis_index("s") * _NC + lax.axis_index("c")
        base = wid * ew

        def chunk(i, carry):
            off = base + i * blk
            pltpu.sync_copy(src_hbm.at[pl.ds(off, blk)], sidx)
            pltpu.sync_copy(dst_hbm.at[pl.ds(off, blk)], didx)
            cp_p = pltpu.async_copy(p_hbm.at[sidx], prow, sem_p)
            cp_d = pltpu.async_copy(dn_hbm.at[didx], drow, sem_d)
            cp_p.wait()
            pltpu.sync_copy(prow, gp_hbm.at[pl.ds(off, blk)])
            cp_d.wait()
            pltpu.sync_copy(drow, gd_hbm.at[pl.ds(off, blk)])
            return carry

        lax.fori_loop(0, n_chunks, chunk, 0)

    def _as_words(t):
        return lax.bitcast_convert_type(
            t.reshape(n_nodes, dw, 2), jnp.int32)

    gp_w, gd_w = _gather_rows(_as_words(p_tab), _as_words(d_tab), src, dst)
    gp = lax.bitcast_convert_type(gp_w, jnp.bfloat16).reshape(n_edges, d2)
    gd = lax.bitcast_convert_type(gd_w, jnp.bfloat16).reshape(n_edges, d2)

    be = 2000
    n_eblk = n_edges // be
    sums, sqs = pl.pallas_call(
        _stats_body,
        grid=(n_eblk,),
        in_specs=[
            pl.BlockSpec((be, d), lambda i: (i, 0)),
            pl.BlockSpec((be, d2), lambda i: (i, 0)),
            pl.BlockSpec((be, d2), lambda i: (i, 0)),
            pl.BlockSpec((d, d2), lambda i: (0, 0)),
        ],
        out_specs=[pl.BlockSpec((1, d2), lambda i: (0, 0)),
                   pl.BlockSpec((1, d2), lambda i: (0, 0))],
        out_shape=[jax.ShapeDtypeStruct((1, d2), jnp.float32),
                   jax.ShapeDtypeStruct((1, d2), jnp.float32)],
    )(edge_attr, gp, gd, w_edge)

    m = pl.pallas_call(
        functools.partial(_act_body, float(n_edges), d),
        grid=(n_eblk,),
        in_specs=[
            pl.BlockSpec((be, d), lambda i: (i, 0)),
            pl.BlockSpec((be, d2), lambda i: (i, 0)),
            pl.BlockSpec((be, d2), lambda i: (i, 0)),
            pl.BlockSpec((d, d2), lambda i: (0, 0)),
            pl.BlockSpec((1, d2), lambda i: (0, 0)),
            pl.BlockSpec((1, d2), lambda i: (0, 0)),
            pl.BlockSpec((1, d2), lambda i: (0, 0)),
            pl.BlockSpec((1, d2), lambda i: (0, 0)),
        ],
        out_specs=pl.BlockSpec((be, d), lambda i: (i, 0)),
        out_shape=jax.ShapeDtypeStruct((n_edges, d), jnp.float32),
    )(edge_attr, gp, gd, w_edge, sums, sqs, gam, bet)

    n_pad = ((n_nodes + 8 * _NS - 1) // (8 * _NS)) * (8 * _NS)
    rows_per_tile = n_pad // _NS

    @functools.partial(
        pl.kernel,
        out_type=jax.ShapeDtypeStruct((_NC, n_pad, d), jnp.float32),
        mesh=mesh,
        scratch_types=[
            pltpu.VMEM((blk,), jnp.int32),
            pltpu.VMEM((blk, d), jnp.float32),
            pltpu.VMEM((64, d), jnp.float32),
            pltpu.VMEM_SHARED((n_pad, d), jnp.float32),
            pltpu.SemaphoreType.DMA,
        ],
    )
    def _scatter_add(m_hbm, dst_hbm, out_hbm, didx, mrow, zbuf, agg_sh, sem):
        c = lax.axis_index("c")
        s = lax.axis_index("s")
        wid = s * _NC + c

        def zrow(r, carry):
            for j in range(d // _LANES):
                zbuf[r, pl.ds(j * _LANES, _LANES)] = jnp.zeros(
                    (_LANES,), jnp.float32)
            return carry

        lax.fori_loop(0, 64, zrow, 0)
        n_zc = (rows_per_tile + 63) // 64

        def zcopy(i, carry):
            off = jnp.minimum(i * 64, rows_per_tile - 64)
            pltpu.sync_copy(zbuf,
                            agg_sh.at[pl.ds(s * rows_per_tile + off, 64)])
            return carry

        lax.fori_loop(0, n_zc, zcopy, 0)
        plsc.subcore_barrier()

        base = wid * ew

        def chunk(i, carry):
            off = base + i * blk
            pltpu.sync_copy(dst_hbm.at[pl.ds(off, blk)], didx)
            pltpu.sync_copy(m_hbm.at[pl.ds(off, blk)], mrow)
            pltpu.sync_copy(mrow, agg_sh.at[didx], add=True)
            return carry

        lax.fori_loop(0, n_chunks, chunk, 0)
        plsc.subcore_barrier()
        pltpu.sync_copy(
            agg_sh.at[pl.ds(s * rows_per_tile, rows_per_tile)],
            out_hbm.at[c, pl.ds(s * rows_per_tile, rows_per_tile)])

    partials = _scatter_add(m, dst)

    node_out = pl.pallas_call(
        functools.partial(_final_body, n_nodes),
        out_shape=jax.ShapeDtypeStruct((n_nodes, d), jnp.float32),
    )(partials, x, gamma_node.reshape(1, d), beta_node.reshape(1, d))

    return (node_out, edge_attr)


# speedup vs baseline: 3.7293x; 3.7293x over previous
import functools

import jax
import jax.numpy as jnp
from jax import lax
from jax.experimental import pallas as pl
from jax.experimental.pallas import tpu as pltpu
from jax.experimental.pallas import tpu_sc as plsc

_NC = 2
_NS = 16
_NW = _NC * _NS
_EPS = 1e-5
_HI = -65536


def _pack_planes(a, b):
    ai = lax.bitcast_convert_type(a, jnp.int32)
    bi = lax.bitcast_convert_type(b, jnp.int32)
    return (bi & _HI) | lax.shift_right_logical(ai, 16)


def _unpack_planes(w):
    a = lax.bitcast_convert_type(lax.shift_left(w, 16), jnp.float32)
    b = lax.bitcast_convert_type(w & _HI, jnp.float32)
    return a, b


def _proj_body(d_feat, x_ref, ws_ref, wd_ref, p_ref, dn_ref):
    xv = x_ref[...]
    p = jnp.dot(xv, ws_ref[...], preferred_element_type=jnp.float32)
    dn = jnp.dot(xv, wd_ref[...], preferred_element_type=jnp.float32)
    p_ref[...] = _pack_planes(p[:, :d_feat], p[:, d_feat:])
    dn_ref[...] = _pack_planes(dn[:, :d_feat], dn[:, d_feat:])


def _edge_y(d_feat, ea_ref, gp_ref, gd_ref, we_ref):
    pm, pg = _unpack_planes(gp_ref[...])
    dm, dg = _unpack_planes(gd_ref[...])
    a = jnp.dot(ea_ref[...], we_ref[...], preferred_element_type=jnp.float32)
    return pm + dm + a[:, :d_feat], pg + dg + a[:, d_feat:]


def _stats_body(d_feat, ea_ref, gp_ref, gd_ref, we_ref, sum_ref, sq_ref):
    ym, yg = _edge_y(d_feat, ea_ref, gp_ref, gd_ref, we_ref)
    s = jnp.concatenate([jnp.sum(ym, axis=0, keepdims=True),
                         jnp.sum(yg, axis=0, keepdims=True)], axis=1)
    q = jnp.concatenate([jnp.sum(ym * ym, axis=0, keepdims=True),
                         jnp.sum(yg * yg, axis=0, keepdims=True)], axis=1)

    @pl.when(pl.program_id(0) == 0)
    def _():
        sum_ref[...] = s
        sq_ref[...] = q

    @pl.when(pl.program_id(0) > 0)
    def _():
        sum_ref[...] += s
        sq_ref[...] += q


def _act_body(n_edges, d_feat, ea_ref, gp_ref, gd_ref, we_ref, sum_ref,
              sq_ref, gam_ref, bet_ref, m_ref):
    inv_e = 1.0 / n_edges
    mean = sum_ref[...] * inv_e
    var = sq_ref[...] * inv_e - mean * mean
    inv = lax.rsqrt(var + _EPS)
    scale = gam_ref[...] * inv
    shift = bet_ref[...] - mean * scale
    ym, yg = _edge_y(d_feat, ea_ref, gp_ref, gd_ref, we_ref)
    zm = ym * scale[:, :d_feat] + shift[:, :d_feat]
    zg = yg * scale[:, d_feat:] + shift[:, d_feat:]
    m_ref[...] = jax.nn.sigmoid(zm) * jax.nn.softplus(zg)


def _final_body(n_nodes, part_ref, x_ref, gn_ref, bn_ref, out_ref):
    agg = part_ref[0, :n_nodes, :] + part_ref[1, :n_nodes, :]
    mean = jnp.mean(agg, axis=0, keepdims=True)
    cent = agg - mean
    var = jnp.mean(cent * cent, axis=0, keepdims=True)
    z = cent * lax.rsqrt(var + _EPS) * gn_ref[...] + bn_ref[...]
    out_ref[...] = jax.nn.sigmoid(z + x_ref[...])


def kernel(x, edge_index, edge_attr, W_mlpt, b_mlpt, gamma_mlpt, beta_mlpt,
           W_gate, b_gate, gamma_gate, beta_gate, gamma_node, beta_node):
    del b_mlpt, b_gate
    n_nodes, d = x.shape
    n_edges = edge_attr.shape[0]
    d2 = 2 * d

    w_src = jnp.concatenate([W_mlpt[:d], W_gate[:d]], axis=1)
    w_dst = jnp.concatenate([W_mlpt[d:2 * d], W_gate[d:2 * d]], axis=1)
    w_edge = jnp.concatenate([W_mlpt[2 * d:], W_gate[2 * d:]], axis=1)
    gam = jnp.concatenate([gamma_mlpt, gamma_gate]).reshape(1, d2)
    bet = jnp.concatenate([beta_mlpt, beta_gate]).reshape(1, d2)
    src = edge_index[0]
    dst = edge_index[1]

    p_tab, d_tab = pl.pallas_call(
        functools.partial(_proj_body, d),
        out_shape=[jax.ShapeDtypeStruct((n_nodes, d), jnp.int32),
                   jax.ShapeDtypeStruct((n_nodes, d), jnp.int32)],
    )(x, w_src, w_dst)

    ew = n_edges // _NW
    blk = 80
    n_chunks = ew // blk
    n_pairs = (n_chunks + 1) // 2
    mesh = plsc.VectorSubcoreMesh(core_axis_name="c", subcore_axis_name="s",
                                  num_cores=_NC, num_subcores=_NS)

    @functools.partial(
        pl.kernel,
        out_type=[jax.ShapeDtypeStruct((n_edges, d), jnp.int32),
                  jax.ShapeDtypeStruct((n_edges, d), jnp.int32)],
        mesh=mesh,
        scratch_types=[
            pltpu.VMEM((blk,), jnp.int32), pltpu.VMEM((blk,), jnp.int32),
            pltpu.VMEM((blk,), jnp.int32), pltpu.VMEM((blk,), jnp.int32),
            pltpu.VMEM((blk, d), jnp.int32), pltpu.VMEM((blk, d), jnp.int32),
            pltpu.VMEM((blk, d), jnp.int32), pltpu.VMEM((blk, d), jnp.int32),
            pltpu.SemaphoreType.DMA, pltpu.SemaphoreType.DMA,
            pltpu.SemaphoreType.DMA, pltpu.SemaphoreType.DMA,
        ],
    )
    def _gather_rows(p_hbm, dn_hbm, src_hbm, dst_hbm, gp_hbm, gd_hbm,
                     sidx0, sidx1, didx0, didx1, prow0, prow1, drow0, drow1,
                     semp0, semp1, semd0, semd1):
        wid = lax.axis_index("s") * _NC + lax.axis_index("c")
        base = wid * ew
        sidx = (sidx0, sidx1)
        didx = (didx0, didx1)
        prow = (prow0, prow1)
        drow = (drow0, drow1)
        semp = (semp0, semp1)
        semd = (semd0, semd1)

        def start(i, b):
            off = base + i * blk
            pltpu.sync_copy(src_hbm.at[pl.ds(off, blk)], sidx[b])
            pltpu.sync_copy(dst_hbm.at[pl.ds(off, blk)], didx[b])
            pltpu.async_copy(p_hbm.at[sidx[b]], prow[b], semp[b])
            pltpu.async_copy(dn_hbm.at[didx[b]], drow[b], semd[b])

        def drain(i, b):
            off = base + i * blk
            pltpu.make_async_copy(p_hbm.at[sidx[b]], prow[b], semp[b]).wait()
            pltpu.sync_copy(prow[b], gp_hbm.at[pl.ds(off, blk)])
            pltpu.make_async_copy(dn_hbm.at[didx[b]], drow[b], semd[b]).wait()
            pltpu.sync_copy(drow[b], gd_hbm.at[pl.ds(off, blk)])

        start(0, 0)

        @pl.when(n_chunks > 1)
        def _():
            start(1, 1)

        def pair(pj, carry):
            for b in range(2):
                i = 2 * pj + b

                @pl.when(i < n_chunks)
                def _():
                    drain(i, b)

                    @pl.when(i + 2 < n_chunks)
                    def _():
                        start(i + 2, b)
            return carry

        lax.fori_loop(0, n_pairs, pair, 0)

    gp_w, gd_w = _gather_rows(p_tab, d_tab, src, dst)

    be = 2000
    n_eblk = n_edges // be
    sums, sqs = pl.pallas_call(
        functools.partial(_stats_body, d),
        grid=(n_eblk,),
        in_specs=[
            pl.BlockSpec((be, d), lambda i: (i, 0)),
            pl.BlockSpec((be, d), lambda i: (i, 0)),
            pl.BlockSpec((be, d), lambda i: (i, 0)),
            pl.BlockSpec((d, d2), lambda i: (0, 0)),
        ],
        out_specs=[pl.BlockSpec((1, d2), lambda i: (0, 0)),
                   pl.BlockSpec((1, d2), lambda i: (0, 0))],
        out_shape=[jax.ShapeDtypeStruct((1, d2), jnp.float32),
                   jax.ShapeDtypeStruct((1, d2), jnp.float32)],
    )(edge_attr, gp_w, gd_w, w_edge)

    m = pl.pallas_call(
        functools.partial(_act_body, float(n_edges), d),
        grid=(n_eblk,),
        in_specs=[
            pl.BlockSpec((be, d), lambda i: (i, 0)),
            pl.BlockSpec((be, d), lambda i: (i, 0)),
            pl.BlockSpec((be, d), lambda i: (i, 0)),
            pl.BlockSpec((d, d2), lambda i: (0, 0)),
            pl.BlockSpec((1, d2), lambda i: (0, 0)),
            pl.BlockSpec((1, d2), lambda i: (0, 0)),
            pl.BlockSpec((1, d2), lambda i: (0, 0)),
            pl.BlockSpec((1, d2), lambda i: (0, 0)),
        ],
        out_specs=pl.BlockSpec((be, d), lambda i: (i, 0)),
        out_shape=jax.ShapeDtypeStruct((n_edges, d), jnp.float32),
    )(edge_attr, gp_w, gd_w, w_edge, sums, sqs, gam, bet)

    n_pad = ((n_nodes + 8 * _NS - 1) // (8 * _NS)) * (8 * _NS)
    rows_per_tile = n_pad // _NS

    @functools.partial(
        pl.kernel,
        out_type=jax.ShapeDtypeStruct((_NC, n_pad, d), jnp.float32),
        mesh=mesh,
        scratch_types=[
            pltpu.VMEM((blk,), jnp.int32), pltpu.VMEM((blk,), jnp.int32),
            pltpu.VMEM((blk, d), jnp.float32),
            pltpu.VMEM((blk, d), jnp.float32),
            pltpu.VMEM((64, d), jnp.float32),
            pltpu.VMEM_SHARED((n_pad, d), jnp.float32),
            pltpu.SemaphoreType.DMA, pltpu.SemaphoreType.DMA,
        ],
    )
    def _scatter_add(m_hbm, dst_hbm, out_hbm, didx0, didx1, mrow0, mrow1,
                     zbuf, agg_sh, sem0, sem1):
        c = lax.axis_index("c")
        s = lax.axis_index("s")
        wid = s * _NC + c
        didx = (didx0, didx1)
        mrow = (mrow0, mrow1)
        sem = (sem0, sem1)

        def zrow(r, carry):
            for j in range(d // 16):
                zbuf[r, pl.ds(j * 16, 16)] = jnp.zeros((16,), jnp.float32)
            return carry

        lax.fori_loop(0, 64, zrow, 0)
        n_zc = (rows_per_tile + 63) // 64

        def zcopy(i, carry):
            off = jnp.minimum(i * 64, rows_per_tile - 64)
            pltpu.sync_copy(zbuf,
                            agg_sh.at[pl.ds(s * rows_per_tile + off, 64)])
            return carry

        lax.fori_loop(0, n_zc, zcopy, 0)
        plsc.subcore_barrier()

        base = wid * ew

        def start(i, b):
            off = base + i * blk
            pltpu.sync_copy(dst_hbm.at[pl.ds(off, blk)], didx[b])
            pltpu.async_copy(m_hbm.at[pl.ds(off, blk)], mrow[b], sem[b])

        def drain(i, b):
            off = base + i * blk
            pltpu.make_async_copy(
                m_hbm.at[pl.ds(off, blk)], mrow[b], sem[b]).wait()
            pltpu.sync_copy(mrow[b], agg_sh.at[didx[b]], add=True)

        start(0, 0)

        @pl.when(n_chunks > 1)
        def _():
            start(1, 1)

        def pair(pj, carry):
            for b in range(2):
                i = 2 * pj + b

                @pl.when(i < n_chunks)
                def _():
                    drain(i, b)

                    @pl.when(i + 2 < n_chunks)
                    def _():
                        start(i + 2, b)
            return carry

        lax.fori_loop(0, n_pairs, pair, 0)
        plsc.subcore_barrier()
        pltpu.sync_copy(
            agg_sh.at[pl.ds(s * rows_per_tile, rows_per_tile)],
            out_hbm.at[c, pl.ds(s * rows_per_tile, rows_per_tile)])

    partials = _scatter_add(m, dst)

    node_out = pl.pallas_call(
        functools.partial(_final_body, n_nodes),
        out_shape=jax.ShapeDtypeStruct((n_nodes, d), jnp.float32),
    )(partials, x, gamma_node.reshape(1, d), beta_node.reshape(1, d))

    return (node_out, edge_attr)


# speedup vs baseline: 4.6487x; 1.2465x over previous
import functools

import jax
import jax.numpy as jnp
from jax import lax
from jax.experimental import pallas as pl
from jax.experimental.pallas import tpu as pltpu
from jax.experimental.pallas import tpu_sc as plsc

_NC = 2
_NS = 16
_NW = _NC * _NS
_EPS = 1e-5
_HI = -65536


def _pack_planes(a, b):
    ai = lax.bitcast_convert_type(a, jnp.int32)
    bi = lax.bitcast_convert_type(b, jnp.int32)
    return (bi & _HI) | lax.shift_right_logical(ai, 16)


def _unpack_planes(w):
    a = lax.bitcast_convert_type(lax.shift_left(w, 16), jnp.float32)
    b = lax.bitcast_convert_type(w & _HI, jnp.float32)
    return a, b


def _proj_body(d_feat, x_ref, ws_ref, wd_ref, p_ref, dn_ref):
    xv = x_ref[...]
    p = jnp.dot(xv, ws_ref[...], preferred_element_type=jnp.float32)
    dn = jnp.dot(xv, wd_ref[...], preferred_element_type=jnp.float32)
    p_ref[...] = _pack_planes(p[:, :d_feat], p[:, d_feat:])
    dn_ref[...] = _pack_planes(dn[:, :d_feat], dn[:, d_feat:])


def _edge_y(d_feat, ea_ref, gp_ref, gd_ref, we_ref):
    pm, pg = _unpack_planes(gp_ref[...])
    dm, dg = _unpack_planes(gd_ref[...])
    a = jnp.dot(ea_ref[...], we_ref[...], preferred_element_type=jnp.float32)
    return pm + dm + a[:, :d_feat], pg + dg + a[:, d_feat:]


def _stats_body(d_feat, ea_ref, gp_ref, gd_ref, we_ref, sum_ref, sq_ref,
                yw_ref):
    ym, yg = _edge_y(d_feat, ea_ref, gp_ref, gd_ref, we_ref)
    yw_ref[...] = _pack_planes(ym, yg)
    s = jnp.concatenate([jnp.sum(ym, axis=0, keepdims=True),
                         jnp.sum(yg, axis=0, keepdims=True)], axis=1)
    q = jnp.concatenate([jnp.sum(ym * ym, axis=0, keepdims=True),
                         jnp.sum(yg * yg, axis=0, keepdims=True)], axis=1)

    @pl.when(pl.program_id(0) == 0)
    def _():
        sum_ref[...] = s
        sq_ref[...] = q

    @pl.when(pl.program_id(0) > 0)
    def _():
        sum_ref[...] += s
        sq_ref[...] += q


def _act_body(n_edges, d_feat, yw_ref, sum_ref, sq_ref, gam_ref, bet_ref,
              m_ref):
    inv_e = 1.0 / n_edges
    mean = sum_ref[...] * inv_e
    var = sq_ref[...] * inv_e - mean * mean
    inv = lax.rsqrt(var + _EPS)
    scale = gam_ref[...] * inv
    shift = bet_ref[...] - mean * scale
    ym, yg = _unpack_planes(yw_ref[...])
    zm = ym * scale[:, :d_feat] + shift[:, :d_feat]
    zg = yg * scale[:, d_feat:] + shift[:, d_feat:]
    m_ref[...] = jax.nn.sigmoid(zm) * jax.nn.softplus(zg)


def _final_body(n_nodes, part_ref, x_ref, gn_ref, bn_ref, out_ref):
    agg = part_ref[0, :n_nodes, :] + part_ref[1, :n_nodes, :]
    mean = jnp.mean(agg, axis=0, keepdims=True)
    cent = agg - mean
    var = jnp.mean(cent * cent, axis=0, keepdims=True)
    z = cent * lax.rsqrt(var + _EPS) * gn_ref[...] + bn_ref[...]
    out_ref[...] = jax.nn.sigmoid(z + x_ref[...])


def kernel(x, edge_index, edge_attr, W_mlpt, b_mlpt, gamma_mlpt, beta_mlpt,
           W_gate, b_gate, gamma_gate, beta_gate, gamma_node, beta_node):
    del b_mlpt, b_gate
    n_nodes, d = x.shape
    n_edges = edge_attr.shape[0]
    d2 = 2 * d

    w_src = jnp.concatenate([W_mlpt[:d], W_gate[:d]], axis=1)
    w_dst = jnp.concatenate([W_mlpt[d:2 * d], W_gate[d:2 * d]], axis=1)
    w_edge = jnp.concatenate([W_mlpt[2 * d:], W_gate[2 * d:]], axis=1)
    gam = jnp.concatenate([gamma_mlpt, gamma_gate]).reshape(1, d2)
    bet = jnp.concatenate([beta_mlpt, beta_gate]).reshape(1, d2)
    src = edge_index[0]
    dst = edge_index[1]

    p_tab, d_tab = pl.pallas_call(
        functools.partial(_proj_body, d),
        out_shape=[jax.ShapeDtypeStruct((n_nodes, d), jnp.int32),
                   jax.ShapeDtypeStruct((n_nodes, d), jnp.int32)],
    )(x, w_src, w_dst)

    ew = n_edges // _NW
    blk = 80
    n_chunks = ew // blk
    nbuf = 3
    n_groups = (n_chunks + nbuf - 1) // nbuf
    mesh = plsc.VectorSubcoreMesh(core_axis_name="c", subcore_axis_name="s",
                                  num_cores=_NC, num_subcores=_NS)

    @functools.partial(
        pl.kernel,
        out_type=[jax.ShapeDtypeStruct((n_edges, d), jnp.int32),
                  jax.ShapeDtypeStruct((n_edges, d), jnp.int32)],
        mesh=mesh,
        scratch_types=[
            [pltpu.VMEM((blk,), jnp.int32)] * nbuf,
            [pltpu.VMEM((blk,), jnp.int32)] * nbuf,
            [pltpu.VMEM((blk, d), jnp.int32)] * nbuf,
            [pltpu.VMEM((blk, d), jnp.int32)] * nbuf,
            [pltpu.SemaphoreType.DMA] * nbuf,
            [pltpu.SemaphoreType.DMA] * nbuf,
            [pltpu.SemaphoreType.DMA] * nbuf,
            [pltpu.SemaphoreType.DMA] * nbuf,
        ],
    )
    def _gather_rows(p_hbm, dn_hbm, src_hbm, dst_hbm, gp_hbm, gd_hbm,
                     sidx, didx, prow, drow, semp, semd, semwp, semwd):
        wid = lax.axis_index("s") * _NC + lax.axis_index("c")
        base = wid * ew

        def start(i, b):
            off = base + i * blk
            pltpu.sync_copy(src_hbm.at[pl.ds(off, blk)], sidx[b])
            pltpu.sync_copy(dst_hbm.at[pl.ds(off, blk)], didx[b])
            pltpu.async_copy(p_hbm.at[sidx[b]], prow[b], semp[b])
            pltpu.async_copy(dn_hbm.at[didx[b]], drow[b], semd[b])

        def drain(i, b):
            off = base + i * blk
            pltpu.make_async_copy(p_hbm.at[sidx[b]], prow[b], semp[b]).wait()
            pltpu.async_copy(prow[b], gp_hbm.at[pl.ds(off, blk)], semwp[b])
            pltpu.make_async_copy(dn_hbm.at[didx[b]], drow[b], semd[b]).wait()
            pltpu.async_copy(drow[b], gd_hbm.at[pl.ds(off, blk)], semwd[b])

        def wait_wb(i, b):
            off = base + i * blk
            pltpu.make_async_copy(
                prow[b], gp_hbm.at[pl.ds(off, blk)], semwp[b]).wait()
            pltpu.make_async_copy(
                drow[b], gd_hbm.at[pl.ds(off, blk)], semwd[b]).wait()

        for b in range(nbuf):
            if b < n_chunks:
                start(b, b)

        def group(gj, carry):
            for b in range(nbuf):
                i = gj * nbuf + b

                @pl.when(i < n_chunks)
                def _():
                    drain(i, b)
            for b in range(nbuf):
                i = gj * nbuf + b

                @pl.when(i + nbuf < n_chunks)
                def _():
                    wait_wb(i, b)
                    start(i + nbuf, b)
            return carry

        lax.fori_loop(0, n_groups, group, 0)
        for i in range(max(0, n_chunks - nbuf), n_chunks):
            wait_wb(i, i % nbuf)

    gp_w, gd_w = _gather_rows(p_tab, d_tab, src, dst)

    be = 4000
    n_eblk = n_edges // be
    sums, sqs, y_w = pl.pallas_call(
        functools.partial(_stats_body, d),
        grid=(n_eblk,),
        in_specs=[
            pl.BlockSpec((be, d), lambda i: (i, 0)),
            pl.BlockSpec((be, d), lambda i: (i, 0)),
            pl.BlockSpec((be, d), lambda i: (i, 0)),
            pl.BlockSpec((d, d2), lambda i: (0, 0)),
        ],
        out_specs=[pl.BlockSpec((1, d2), lambda i: (0, 0)),
                   pl.BlockSpec((1, d2), lambda i: (0, 0)),
                   pl.BlockSpec((be, d), lambda i: (i, 0))],
        out_shape=[jax.ShapeDtypeStruct((1, d2), jnp.float32),
                   jax.ShapeDtypeStruct((1, d2), jnp.float32),
                   jax.ShapeDtypeStruct((n_edges, d), jnp.int32)],
    )(edge_attr, gp_w, gd_w, w_edge)

    m = pl.pallas_call(
        functools.partial(_act_body, float(n_edges), d),
        grid=(n_eblk,),
        in_specs=[
            pl.BlockSpec((be, d), lambda i: (i, 0)),
            pl.BlockSpec((1, d2), lambda i: (0, 0)),
            pl.BlockSpec((1, d2), lambda i: (0, 0)),
            pl.BlockSpec((1, d2), lambda i: (0, 0)),
            pl.BlockSpec((1, d2), lambda i: (0, 0)),
        ],
        out_specs=pl.BlockSpec((be, d), lambda i: (i, 0)),
        out_shape=jax.ShapeDtypeStruct((n_edges, d), jnp.float32),
    )(y_w, sums, sqs, gam, bet)

    n_pad = ((n_nodes + 8 * _NS - 1) // (8 * _NS)) * (8 * _NS)
    rows_per_tile = n_pad // _NS

    @functools.partial(
        pl.kernel,
        out_type=jax.ShapeDtypeStruct((_NC, n_pad, d), jnp.float32),
        mesh=mesh,
        scratch_types=[
            [pltpu.VMEM((blk,), jnp.int32)] * nbuf,
            [pltpu.VMEM((blk, d), jnp.float32)] * nbuf,
            pltpu.VMEM((64, d), jnp.float32),
            pltpu.VMEM_SHARED((n_pad, d), jnp.float32),
            [pltpu.SemaphoreType.DMA] * nbuf,
            [pltpu.SemaphoreType.DMA] * nbuf,
        ],
    )
    def _scatter_add(m_hbm, dst_hbm, out_hbm, didx, mrow, zbuf, agg_sh,
                     semm, semx):
        c = lax.axis_index("c")
        s = lax.axis_index("s")
        wid = s * _NC + c

        def zrow(r, carry):
            for j in range(d // 16):
                zbuf[r, pl.ds(j * 16, 16)] = jnp.zeros((16,), jnp.float32)
            return carry

        lax.fori_loop(0, 64, zrow, 0)
        n_zc = (rows_per_tile + 63) // 64

        def zcopy(i, carry):
            off = jnp.minimum(i * 64, rows_per_tile - 64)
            pltpu.sync_copy(zbuf,
                            agg_sh.at[pl.ds(s * rows_per_tile + off, 64)])
            return carry

        lax.fori_loop(0, n_zc, zcopy, 0)
        plsc.subcore_barrier()

        base = wid * ew

        def start(i, b):
            off = base + i * blk
            pltpu.async_copy(dst_hbm.at[pl.ds(off, blk)], didx[b], semx[b])
            pltpu.async_copy(m_hbm.at[pl.ds(off, blk)], mrow[b], semm[b])

        def drain(i, b):
            off = base + i * blk
            pltpu.make_async_copy(
                dst_hbm.at[pl.ds(off, blk)], didx[b], semx[b]).wait()
            pltpu.make_async_copy(
                m_hbm.at[pl.ds(off, blk)], mrow[b], semm[b]).wait()
            pltpu.sync_copy(mrow[b], agg_sh.at[didx[b]], add=True)

        for b in range(nbuf):
            if b < n_chunks:
                start(b, b)

        def group(gj, carry):
            for b in range(nbuf):
                i = gj * nbuf + b

                @pl.when(i < n_chunks)
                def _():
                    drain(i, b)

                    @pl.when(i + nbuf < n_chunks)
                    def _():
                        start(i + nbuf, b)
            return carry

        lax.fori_loop(0, n_groups, group, 0)
        plsc.subcore_barrier()
        pltpu.sync_copy(
            agg_sh.at[pl.ds(s * rows_per_tile, rows_per_tile)],
            out_hbm.at[c, pl.ds(s * rows_per_tile, rows_per_tile)])

    partials = _scatter_add(m, dst)

    node_out = pl.pallas_call(
        functools.partial(_final_body, n_nodes),
        out_shape=jax.ShapeDtypeStruct((n_nodes, d), jnp.float32),
    )(partials, x, gamma_node.reshape(1, d), beta_node.reshape(1, d))

    return (node_out, edge_attr)


# speedup vs baseline: 4.6523x; 1.0008x over previous
import functools

import jax
import jax.numpy as jnp
from jax import lax
from jax.experimental import pallas as pl
from jax.experimental.pallas import tpu as pltpu
from jax.experimental.pallas import tpu_sc as plsc

_NC = 2
_NS = 16
_NW = _NC * _NS
_EPS = 1e-5
_HI = -65536


def _pack_planes(a, b):
    ai = lax.bitcast_convert_type(a, jnp.int32)
    bi = lax.bitcast_convert_type(b, jnp.int32)
    return (bi & _HI) | lax.shift_right_logical(ai, 16)


def _unpack_planes(w):
    a = lax.bitcast_convert_type(lax.shift_left(w, 16), jnp.float32)
    b = lax.bitcast_convert_type(w & _HI, jnp.float32)
    return a, b


def _proj_body(d_feat, x_ref, ws_ref, wd_ref, p_ref, dn_ref):
    xv = x_ref[...]
    p = jnp.dot(xv, ws_ref[...], preferred_element_type=jnp.float32)
    dn = jnp.dot(xv, wd_ref[...], preferred_element_type=jnp.float32)
    p_ref[...] = _pack_planes(p[:, :d_feat], p[:, d_feat:])
    dn_ref[...] = _pack_planes(dn[:, :d_feat], dn[:, d_feat:])


def _stats_body(d_feat, ea_ref, g_ref, we_ref, sum_ref, sq_ref, yw_ref):
    gm, gg = _unpack_planes(g_ref[...])
    a = jnp.dot(ea_ref[...], we_ref[...], preferred_element_type=jnp.float32)
    ym = gm + a[:, :d_feat]
    yg = gg + a[:, d_feat:]
    yw_ref[...] = _pack_planes(ym, yg)
    s = jnp.concatenate([jnp.sum(ym, axis=0, keepdims=True),
                         jnp.sum(yg, axis=0, keepdims=True)], axis=1)
    q = jnp.concatenate([jnp.sum(ym * ym, axis=0, keepdims=True),
                         jnp.sum(yg * yg, axis=0, keepdims=True)], axis=1)

    @pl.when(pl.program_id(0) == 0)
    def _():
        sum_ref[...] = s
        sq_ref[...] = q

    @pl.when(pl.program_id(0) > 0)
    def _():
        sum_ref[...] += s
        sq_ref[...] += q


def _act_body(n_edges, d_feat, yw_ref, sum_ref, sq_ref, gam_ref, bet_ref,
              m_ref):
    inv_e = 1.0 / n_edges
    mean = sum_ref[...] * inv_e
    var = sq_ref[...] * inv_e - mean * mean
    inv = lax.rsqrt(var + _EPS)
    scale = gam_ref[...] * inv
    shift = bet_ref[...] - mean * scale
    ym, yg = _unpack_planes(yw_ref[...])
    zm = ym * scale[:, :d_feat] + shift[:, :d_feat]
    zg = yg * scale[:, d_feat:] + shift[:, d_feat:]
    m_ref[...] = jax.nn.sigmoid(zm) * jax.nn.softplus(zg)


def _final_body(n_nodes, part_ref, x_ref, gn_ref, bn_ref, out_ref):
    agg = part_ref[0, :n_nodes, :] + part_ref[1, :n_nodes, :]
    mean = jnp.mean(agg, axis=0, keepdims=True)
    cent = agg - mean
    var = jnp.mean(cent * cent, axis=0, keepdims=True)
    z = cent * lax.rsqrt(var + _EPS) * gn_ref[...] + bn_ref[...]
    out_ref[...] = jax.nn.sigmoid(z + x_ref[...])


def kernel(x, edge_index, edge_attr, W_mlpt, b_mlpt, gamma_mlpt, beta_mlpt,
           W_gate, b_gate, gamma_gate, beta_gate, gamma_node, beta_node):
    del b_mlpt, b_gate
    n_nodes, d = x.shape
    n_edges = edge_attr.shape[0]
    d2 = 2 * d

    w_src = jnp.concatenate([W_mlpt[:d], W_gate[:d]], axis=1)
    w_dst = jnp.concatenate([W_mlpt[d:2 * d], W_gate[d:2 * d]], axis=1)
    w_edge = jnp.concatenate([W_mlpt[2 * d:], W_gate[2 * d:]], axis=1)
    gam = jnp.concatenate([gamma_mlpt, gamma_gate]).reshape(1, d2)
    bet = jnp.concatenate([beta_mlpt, beta_gate]).reshape(1, d2)
    src = edge_index[0]
    dst = edge_index[1]

    p_tab, d_tab = pl.pallas_call(
        functools.partial(_proj_body, d),
        out_shape=[jax.ShapeDtypeStruct((n_nodes, d), jnp.int32),
                   jax.ShapeDtypeStruct((n_nodes, d), jnp.int32)],
    )(x, w_src, w_dst)

    ew = n_edges // _NW
    blk = 80
    n_chunks = ew // blk
    nbuf = 3
    n_groups = (n_chunks + nbuf - 1) // nbuf
    mesh = plsc.VectorSubcoreMesh(core_axis_name="c", subcore_axis_name="s",
                                  num_cores=_NC, num_subcores=_NS)

    @functools.partial(
        pl.kernel,
        out_type=jax.ShapeDtypeStruct((n_edges, d), jnp.int32),
        mesh=mesh,
        scratch_types=[
            [pltpu.VMEM((blk,), jnp.int32)] * nbuf,
            [pltpu.VMEM((blk,), jnp.int32)] * nbuf,
            [pltpu.VMEM((blk, d), jnp.int32)] * nbuf,
            [pltpu.VMEM((blk, d), jnp.int32)] * nbuf,
            [pltpu.SemaphoreType.DMA] * nbuf,
            [pltpu.SemaphoreType.DMA] * nbuf,
            [pltpu.SemaphoreType.DMA] * nbuf,
        ],
    )
    def _gather_add(p_hbm, dn_hbm, src_hbm, dst_hbm, g_hbm,
                    sidx, didx, prow, drow, semp, semd, semw):
        wid = lax.axis_index("s") * _NC + lax.axis_index("c")
        base = wid * ew

        def start(i, b):
            off = base + i * blk
            pltpu.sync_copy(src_hbm.at[pl.ds(off, blk)], sidx[b])
            pltpu.sync_copy(dst_hbm.at[pl.ds(off, blk)], didx[b])
            pltpu.async_copy(p_hbm.at[sidx[b]], prow[b], semp[b])
            pltpu.async_copy(dn_hbm.at[didx[b]], drow[b], semd[b])

        def drain(i, b):
            off = base + i * blk
            pltpu.make_async_copy(p_hbm.at[sidx[b]], prow[b], semp[b]).wait()
            pltpu.make_async_copy(dn_hbm.at[didx[b]], drow[b], semd[b]).wait()
            pb = prow[b]
            db = drow[b]

            def row(r, carry):
                for j in range(d // 16):
                    sl = pl.ds(j * 16, 16)
                    bf = lambda w: lax.bitcast_convert_type(w, jnp.float32)
                    bi = lambda f: lax.bitcast_convert_type(f, jnp.int32)
                    pw = pb[r, sl]
                    dw = db[r, sl]
                    sm = bf(lax.shift_left(pw, 16)) + bf(lax.shift_left(dw, 16))
                    sg = bf(pw & _HI) + bf(dw & _HI)
                    pb[r, sl] = ((bi(sg) & _HI)
                                 | lax.shift_right_logical(bi(sm), 16))
                return carry

            lax.fori_loop(0, blk, row, 0)
            pltpu.async_copy(pb, g_hbm.at[pl.ds(off, blk)], semw[b])

        def wait_wb(i, b):
            off = base + i * blk
            pltpu.make_async_copy(
                prow[b], g_hbm.at[pl.ds(off, blk)], semw[b]).wait()

        for b in range(nbuf):
            if b < n_chunks:
                start(b, b)

        def group(gj, carry):
            for b in range(nbuf):
                i = gj * nbuf + b

                @pl.when(i < n_chunks)
                def _():
                    drain(i, b)
            for b in range(nbuf):
                i = gj * nbuf + b

                @pl.when(i + nbuf < n_chunks)
                def _():
                    wait_wb(i, b)
                    start(i + nbuf, b)
            return carry

        lax.fori_loop(0, n_groups, group, 0)
        for i in range(max(0, n_chunks - nbuf), n_chunks):
            wait_wb(i, i % nbuf)

    g_w = _gather_add(p_tab, d_tab, src, dst)

    be = 4000
    n_eblk = n_edges // be
    sums, sqs, y_w = pl.pallas_call(
        functools.partial(_stats_body, d),
        grid=(n_eblk,),
        in_specs=[
            pl.BlockSpec((be, d), lambda i: (i, 0)),
            pl.BlockSpec((be, d), lambda i: (i, 0)),
            pl.BlockSpec((d, d2), lambda i: (0, 0)),
        ],
        out_specs=[pl.BlockSpec((1, d2), lambda i: (0, 0)),
                   pl.BlockSpec((1, d2), lambda i: (0, 0)),
                   pl.BlockSpec((be, d), lambda i: (i, 0))],
        out_shape=[jax.ShapeDtypeStruct((1, d2), jnp.float32),
                   jax.ShapeDtypeStruct((1, d2), jnp.float32),
                   jax.ShapeDtypeStruct((n_edges, d), jnp.int32)],
    )(edge_attr, g_w, w_edge)

    m = pl.pallas_call(
        functools.partial(_act_body, float(n_edges), d),
        grid=(n_eblk,),
        in_specs=[
            pl.BlockSpec((be, d), lambda i: (i, 0)),
            pl.BlockSpec((1, d2), lambda i: (0, 0)),
            pl.BlockSpec((1, d2), lambda i: (0, 0)),
            pl.BlockSpec((1, d2), lambda i: (0, 0)),
            pl.BlockSpec((1, d2), lambda i: (0, 0)),
        ],
        out_specs=pl.BlockSpec((be, d), lambda i: (i, 0)),
        out_shape=jax.ShapeDtypeStruct((n_edges, d), jnp.float32),
    )(y_w, sums, sqs, gam, bet)

    n_pad = ((n_nodes + 8 * _NS - 1) // (8 * _NS)) * (8 * _NS)
    rows_per_tile = n_pad // _NS

    @functools.partial(
        pl.kernel,
        out_type=jax.ShapeDtypeStruct((_NC, n_pad, d), jnp.float32),
        mesh=mesh,
        scratch_types=[
            [pltpu.VMEM((blk,), jnp.int32)] * nbuf,
            [pltpu.VMEM((blk, d), jnp.float32)] * nbuf,
            pltpu.VMEM((64, d), jnp.float32),
            pltpu.VMEM_SHARED((n_pad, d), jnp.float32),
            [pltpu.SemaphoreType.DMA] * nbuf,
            [pltpu.SemaphoreType.DMA] * nbuf,
        ],
    )
    def _scatter_add(m_hbm, dst_hbm, out_hbm, didx, mrow, zbuf, agg_sh,
                     semm, semx):
        c = lax.axis_index("c")
        s = lax.axis_index("s")
        wid = s * _NC + c

        def zrow(r, carry):
            for j in range(d // 16):
                zbuf[r, pl.ds(j * 16, 16)] = jnp.zeros((16,), jnp.float32)
            return carry

        lax.fori_loop(0, 64, zrow, 0)
        n_zc = (rows_per_tile + 63) // 64

        def zcopy(i, carry):
            off = jnp.minimum(i * 64, rows_per_tile - 64)
            pltpu.sync_copy(zbuf,
                            agg_sh.at[pl.ds(s * rows_per_tile + off, 64)])
            return carry

        lax.fori_loop(0, n_zc, zcopy, 0)
        plsc.subcore_barrier()

        base = wid * ew

        def start(i, b):
            off = base + i * blk
            pltpu.async_copy(dst_hbm.at[pl.ds(off, blk)], didx[b], semx[b])
            pltpu.async_copy(m_hbm.at[pl.ds(off, blk)], mrow[b], semm[b])

        def drain(i, b):
            off = base + i * blk
            pltpu.make_async_copy(
                dst_hbm.at[pl.ds(off, blk)], didx[b], semx[b]).wait()
            pltpu.make_async_copy(
                m_hbm.at[pl.ds(off, blk)], mrow[b], semm[b]).wait()
            pltpu.sync_copy(mrow[b], agg_sh.at[didx[b]], add=True)

        for b in range(nbuf):
            if b < n_chunks:
                start(b, b)

        def group(gj, carry):
            for b in range(nbuf):
                i = gj * nbuf + b

                @pl.when(i < n_chunks)
                def _():
                    drain(i, b)

                    @pl.when(i + nbuf < n_chunks)
                    def _():
                        start(i + nbuf, b)
            return carry

        lax.fori_loop(0, n_groups, group, 0)
        plsc.subcore_barrier()
        pltpu.sync_copy(
            agg_sh.at[pl.ds(s * rows_per_tile, rows_per_tile)],
            out_hbm.at[c, pl.ds(s * rows_per_tile, rows_per_tile)])

    partials = _scatter_add(m, dst)

    node_out = pl.pallas_call(
        functools.partial(_final_body, n_nodes),
        out_shape=jax.ShapeDtypeStruct((n_nodes, d), jnp.float32),
    )(partials, x, gamma_node.reshape(1, d), beta_node.reshape(1, d))

    return (node_out, edge_attr)


# speedup vs baseline: 4.8316x; 1.0386x over previous
import functools

import jax
import jax.numpy as jnp
from jax import lax
from jax.experimental import pallas as pl
from jax.experimental.pallas import tpu as pltpu
from jax.experimental.pallas import tpu_sc as plsc

_NC = 2
_NS = 16
_NW = _NC * _NS
_EPS = 1e-5
_HI = -65536


def _pack_planes(a, b):
    ai = lax.bitcast_convert_type(a, jnp.int32)
    bi = lax.bitcast_convert_type(b, jnp.int32)
    return (bi & _HI) | lax.shift_right_logical(ai, 16)


def _unpack_planes(w):
    a = lax.bitcast_convert_type(lax.shift_left(w, 16), jnp.float32)
    b = lax.bitcast_convert_type(w & _HI, jnp.float32)
    return a, b


def _proj_body(d_feat, x_ref, ws_ref, wd_ref, p_ref, dn_ref):
    xv = x_ref[...]
    p = jnp.dot(xv, ws_ref[...], preferred_element_type=jnp.float32)
    dn = jnp.dot(xv, wd_ref[...], preferred_element_type=jnp.float32)
    p_ref[...] = _pack_planes(p[:, :d_feat], p[:, d_feat:])
    dn_ref[...] = _pack_planes(dn[:, :d_feat], dn[:, d_feat:])


def _stats_body(d_feat, ea_ref, g_ref, we_ref, sum_ref, sq_ref, yw_ref):
    gm, gg = _unpack_planes(g_ref[...])
    a = jnp.dot(ea_ref[...], we_ref[...], preferred_element_type=jnp.float32)
    ym = gm + a[:, :d_feat]
    yg = gg + a[:, d_feat:]
    yw_ref[...] = _pack_planes(ym, yg)
    s = jnp.concatenate([jnp.sum(ym, axis=0, keepdims=True),
                         jnp.sum(yg, axis=0, keepdims=True)], axis=1)
    q = jnp.concatenate([jnp.sum(ym * ym, axis=0, keepdims=True),
                         jnp.sum(yg * yg, axis=0, keepdims=True)], axis=1)

    @pl.when(pl.program_id(0) == 0)
    def _():
        sum_ref[...] = s
        sq_ref[...] = q

    @pl.when(pl.program_id(0) > 0)
    def _():
        sum_ref[...] += s
        sq_ref[...] += q


def _act_body(n_edges, d_feat, yw_ref, sum_ref, sq_ref, gam_ref, bet_ref,
              m_ref):
    inv_e = 1.0 / n_edges
    mean = sum_ref[...] * inv_e
    var = sq_ref[...] * inv_e - mean * mean
    inv = lax.rsqrt(var + _EPS)
    scale = gam_ref[...] * inv
    shift = bet_ref[...] - mean * scale
    ym, yg = _unpack_planes(yw_ref[...])
    zm = ym * scale[:, :d_feat] + shift[:, :d_feat]
    zg = yg * scale[:, d_feat:] + shift[:, d_feat:]
    m_ref[...] = jax.nn.sigmoid(zm) * jax.nn.softplus(zg)


def _final_body(n_nodes, part_ref, x_ref, gn_ref, bn_ref, out_ref):
    agg = part_ref[0, :n_nodes, :] + part_ref[1, :n_nodes, :]
    mean = jnp.mean(agg, axis=0, keepdims=True)
    cent = agg - mean
    var = jnp.mean(cent * cent, axis=0, keepdims=True)
    z = cent * lax.rsqrt(var + _EPS) * gn_ref[...] + bn_ref[...]
    out_ref[...] = jax.nn.sigmoid(z + x_ref[...])


def kernel(x, edge_index, edge_attr, W_mlpt, b_mlpt, gamma_mlpt, beta_mlpt,
           W_gate, b_gate, gamma_gate, beta_gate, gamma_node, beta_node):
    del b_mlpt, b_gate
    n_nodes, d = x.shape
    n_edges = edge_attr.shape[0]
    d2 = 2 * d

    w_src = jnp.concatenate([W_mlpt[:d], W_gate[:d]], axis=1)
    w_dst = jnp.concatenate([W_mlpt[d:2 * d], W_gate[d:2 * d]], axis=1)
    w_edge = jnp.concatenate([W_mlpt[2 * d:], W_gate[2 * d:]], axis=1)
    gam = jnp.concatenate([gamma_mlpt, gamma_gate]).reshape(1, d2)
    bet = jnp.concatenate([beta_mlpt, beta_gate]).reshape(1, d2)
    src = edge_index[0]
    dst = edge_index[1]

    p_tab, d_tab = pl.pallas_call(
        functools.partial(_proj_body, d),
        out_shape=[jax.ShapeDtypeStruct((n_nodes, d), jnp.int32),
                   jax.ShapeDtypeStruct((n_nodes, d), jnp.int32)],
    )(x, w_src, w_dst)

    ew = n_edges // _NW
    blk = 80
    n_chunks = ew // blk
    nbuf = 3
    n_groups = (n_chunks + nbuf - 1) // nbuf
    mesh = plsc.VectorSubcoreMesh(core_axis_name="c", subcore_axis_name="s",
                                  num_cores=_NC, num_subcores=_NS)

    @functools.partial(
        pl.kernel,
        out_type=jax.ShapeDtypeStruct((n_edges, d), jnp.int32),
        mesh=mesh,
        scratch_types=[
            [pltpu.VMEM((blk,), jnp.int32)] * nbuf,
            [pltpu.VMEM((blk,), jnp.int32)] * nbuf,
            [pltpu.VMEM((blk, d), jnp.int32)] * nbuf,
            [pltpu.VMEM((blk, d), jnp.int32)] * nbuf,
            [pltpu.SemaphoreType.DMA] * nbuf,
            [pltpu.SemaphoreType.DMA] * nbuf,
            [pltpu.SemaphoreType.DMA] * nbuf,
        ],
    )
    def _gather_add(p_hbm, dn_hbm, src_hbm, dst_hbm, g_hbm,
                    sidx, didx, prow, drow, semp, semd, semw):
        wid = lax.axis_index("s") * _NC + lax.axis_index("c")
        base = wid * ew

        def start(i, b):
            off = base + i * blk
            pltpu.sync_copy(src_hbm.at[pl.ds(off, blk)], sidx[b])
            pltpu.sync_copy(dst_hbm.at[pl.ds(off, blk)], didx[b])
            pltpu.async_copy(p_hbm.at[sidx[b]], prow[b], semp[b])
            pltpu.async_copy(dn_hbm.at[didx[b]], drow[b], semd[b])

        def drain(i, b):
            off = base + i * blk
            pltpu.make_async_copy(p_hbm.at[sidx[b]], prow[b], semp[b]).wait()
            pltpu.make_async_copy(dn_hbm.at[didx[b]], drow[b], semd[b]).wait()
            pb = prow[b]
            db = drow[b]

            def row(r, carry):
                for j in range(d // 16):
                    sl = pl.ds(j * 16, 16)
                    bf = lambda w: lax.bitcast_convert_type(w, jnp.float32)
                    bi = lambda f: lax.bitcast_convert_type(f, jnp.int32)
                    pw = pb[r, sl]
                    dw = db[r, sl]
                    sm = bf(lax.shift_left(pw, 16)) + bf(lax.shift_left(dw, 16))
                    sg = bf(pw & _HI) + bf(dw & _HI)
                    pb[r, sl] = ((bi(sg) & _HI)
                                 | lax.shift_right_logical(bi(sm), 16))
                return carry

            lax.fori_loop(0, blk, row, 0)
            pltpu.async_copy(pb, g_hbm.at[pl.ds(off, blk)], semw[b])

        def wait_wb(i, b):
            off = base + i * blk
            pltpu.make_async_copy(
                prow[b], g_hbm.at[pl.ds(off, blk)], semw[b]).wait()

        for b in range(nbuf):
            if b < n_chunks:
                start(b, b)

        def group(gj, carry):
            for b in range(nbuf):
                i = gj * nbuf + b

                @pl.when(i < n_chunks)
                def _():
                    drain(i, b)
            for b in range(nbuf):
                i = gj * nbuf + b

                @pl.when(i + nbuf < n_chunks)
                def _():
                    wait_wb(i, b)
                    start(i + nbuf, b)
            return carry

        lax.fori_loop(0, n_groups, group, 0)
        for i in range(max(0, n_chunks - nbuf), n_chunks):
            wait_wb(i, i % nbuf)

    g_w = _gather_add(p_tab, d_tab, src, dst)

    be = 8000
    n_eblk = n_edges // be
    sums, sqs, y_w = pl.pallas_call(
        functools.partial(_stats_body, d),
        grid=(n_eblk,),
        in_specs=[
            pl.BlockSpec((be, d), lambda i: (i, 0)),
            pl.BlockSpec((be, d), lambda i: (i, 0)),
            pl.BlockSpec((d, d2), lambda i: (0, 0)),
        ],
        out_specs=[pl.BlockSpec((1, d2), lambda i: (0, 0)),
                   pl.BlockSpec((1, d2), lambda i: (0, 0)),
                   pl.BlockSpec((be, d), lambda i: (i, 0))],
        out_shape=[jax.ShapeDtypeStruct((1, d2), jnp.float32),
                   jax.ShapeDtypeStruct((1, d2), jnp.float32),
                   jax.ShapeDtypeStruct((n_edges, d), jnp.int32)],
    )(edge_attr, g_w, w_edge)

    m = pl.pallas_call(
        functools.partial(_act_body, float(n_edges), d),
        grid=(n_eblk,),
        in_specs=[
            pl.BlockSpec((be, d), lambda i: (i, 0)),
            pl.BlockSpec((1, d2), lambda i: (0, 0)),
            pl.BlockSpec((1, d2), lambda i: (0, 0)),
            pl.BlockSpec((1, d2), lambda i: (0, 0)),
            pl.BlockSpec((1, d2), lambda i: (0, 0)),
        ],
        out_specs=pl.BlockSpec((be, d), lambda i: (i, 0)),
        out_shape=jax.ShapeDtypeStruct((n_edges, d), jnp.float32),
    )(y_w, sums, sqs, gam, bet)

    n_pad = ((n_nodes + 8 * _NS - 1) // (8 * _NS)) * (8 * _NS)
    rows_per_tile = n_pad // _NS

    @functools.partial(
        pl.kernel,
        out_type=jax.ShapeDtypeStruct((_NC, n_pad, d), jnp.float32),
        mesh=mesh,
        scratch_types=[
            [pltpu.VMEM((blk,), jnp.int32)] * nbuf,
            [pltpu.VMEM((blk, d), jnp.float32)] * nbuf,
            pltpu.VMEM((64, d), jnp.float32),
            pltpu.VMEM_SHARED((n_pad, d), jnp.float32),
            [pltpu.SemaphoreType.DMA] * nbuf,
            [pltpu.SemaphoreType.DMA] * nbuf,
        ],
    )
    def _scatter_add(m_hbm, dst_hbm, out_hbm, didx, mrow, zbuf, agg_sh,
                     semm, semx):
        c = lax.axis_index("c")
        s = lax.axis_index("s")
        wid = s * _NC + c

        def zrow(r, carry):
            for j in range(d // 16):
                zbuf[r, pl.ds(j * 16, 16)] = jnp.zeros((16,), jnp.float32)
            return carry

        lax.fori_loop(0, 64, zrow, 0)
        n_zc = (rows_per_tile + 63) // 64

        def zcopy(i, carry):
            off = jnp.minimum(i * 64, rows_per_tile - 64)
            pltpu.sync_copy(zbuf,
                            agg_sh.at[pl.ds(s * rows_per_tile + off, 64)])
            return carry

        lax.fori_loop(0, n_zc, zcopy, 0)
        plsc.subcore_barrier()

        base = wid * ew

        def start(i, b):
            off = base + i * blk
            pltpu.async_copy(dst_hbm.at[pl.ds(off, blk)], didx[b], semx[b])
            pltpu.async_copy(m_hbm.at[pl.ds(off, blk)], mrow[b], semm[b])

        def drain(i, b):
            off = base + i * blk
            pltpu.make_async_copy(
                dst_hbm.at[pl.ds(off, blk)], didx[b], semx[b]).wait()
            pltpu.make_async_copy(
                m_hbm.at[pl.ds(off, blk)], mrow[b], semm[b]).wait()
            pltpu.sync_copy(mrow[b], agg_sh.at[didx[b]], add=True)

        for b in range(nbuf):
            if b < n_chunks:
                start(b, b)

        def group(gj, carry):
            for b in range(nbuf):
                i = gj * nbuf + b

                @pl.when(i < n_chunks)
                def _():
                    drain(i, b)

                    @pl.when(i + nbuf < n_chunks)
                    def _():
                        start(i + nbuf, b)
            return carry

        lax.fori_loop(0, n_groups, group, 0)
        plsc.subcore_barrier()
        pltpu.sync_copy(
            agg_sh.at[pl.ds(s * rows_per_tile, rows_per_tile)],
            out_hbm.at[c, pl.ds(s * rows_per_tile, rows_per_tile)])

    partials = _scatter_add(m, dst)

    node_out = pl.pallas_call(
        functools.partial(_final_body, n_nodes),
        out_shape=jax.ShapeDtypeStruct((n_nodes, d), jnp.float32),
    )(partials, x, gamma_node.reshape(1, d), beta_node.reshape(1, d))

    return (node_out, edge_attr)


# speedup vs baseline: 4.8556x; 1.0050x over previous
import functools

import jax
import jax.numpy as jnp
from jax import lax
from jax.experimental import pallas as pl
from jax.experimental.pallas import tpu as pltpu
from jax.experimental.pallas import tpu_sc as plsc

_NC = 2
_NS = 16
_NW = _NC * _NS
_EPS = 1e-5
_HI = -65536


def _pack_planes(a, b):
    ai = lax.bitcast_convert_type(a, jnp.int32)
    bi = lax.bitcast_convert_type(b, jnp.int32)
    return (bi & _HI) | lax.shift_right_logical(ai, 16)


def _unpack_planes(w):
    a = lax.bitcast_convert_type(lax.shift_left(w, 16), jnp.float32)
    b = lax.bitcast_convert_type(w & _HI, jnp.float32)
    return a, b


def _proj_body(d_feat, x_ref, ws_ref, wd_ref, p_ref, dn_ref):
    xv = x_ref[...]
    p = jnp.dot(xv, ws_ref[...], preferred_element_type=jnp.float32)
    dn = jnp.dot(xv, wd_ref[...], preferred_element_type=jnp.float32)
    p_ref[...] = _pack_planes(p[:, :d_feat], p[:, d_feat:])
    dn_ref[...] = _pack_planes(dn[:, :d_feat], dn[:, d_feat:])


def _stats_body(d_feat, ea_ref, g_ref, we_ref, sum_ref, sq_ref, yw_ref):
    gm, gg = _unpack_planes(g_ref[...])
    a = jnp.dot(ea_ref[...], we_ref[...], preferred_element_type=jnp.float32)
    ym = gm + a[:, :d_feat]
    yg = gg + a[:, d_feat:]
    yw_ref[...] = _pack_planes(ym, yg)
    s = jnp.concatenate([jnp.sum(ym, axis=0, keepdims=True),
                         jnp.sum(yg, axis=0, keepdims=True)], axis=1)
    q = jnp.concatenate([jnp.sum(ym * ym, axis=0, keepdims=True),
                         jnp.sum(yg * yg, axis=0, keepdims=True)], axis=1)

    @pl.when(pl.program_id(0) == 0)
    def _():
        sum_ref[...] = s
        sq_ref[...] = q

    @pl.when(pl.program_id(0) > 0)
    def _():
        sum_ref[...] += s
        sq_ref[...] += q


def _act_body(n_edges, d_feat, yw_ref, sum_ref, sq_ref, gam_ref, bet_ref,
              m_ref):
    inv_e = 1.0 / n_edges
    mean = sum_ref[...] * inv_e
    var = sq_ref[...] * inv_e - mean * mean
    inv = lax.rsqrt(var + _EPS)
    scale = gam_ref[...] * inv
    shift = bet_ref[...] - mean * scale
    ym, yg = _unpack_planes(yw_ref[...])
    zm = ym * scale[:, :d_feat] + shift[:, :d_feat]
    zg = yg * scale[:, d_feat:] + shift[:, d_feat:]
    m_ref[...] = jax.nn.sigmoid(zm) * jax.nn.softplus(zg)


def _final_body(n_nodes, part_ref, x_ref, gn_ref, bn_ref, out_ref):
    agg = part_ref[0, :n_nodes, :] + part_ref[1, :n_nodes, :]
    mean = jnp.mean(agg, axis=0, keepdims=True)
    cent = agg - mean
    var = jnp.mean(cent * cent, axis=0, keepdims=True)
    z = cent * lax.rsqrt(var + _EPS) * gn_ref[...] + bn_ref[...]
    out_ref[...] = jax.nn.sigmoid(z + x_ref[...])


def kernel(x, edge_index, edge_attr, W_mlpt, b_mlpt, gamma_mlpt, beta_mlpt,
           W_gate, b_gate, gamma_gate, beta_gate, gamma_node, beta_node):
    del b_mlpt, b_gate
    n_nodes, d = x.shape
    n_edges = edge_attr.shape[0]
    d2 = 2 * d

    w_src = jnp.concatenate([W_mlpt[:d], W_gate[:d]], axis=1)
    w_dst = jnp.concatenate([W_mlpt[d:2 * d], W_gate[d:2 * d]], axis=1)
    w_edge = jnp.concatenate([W_mlpt[2 * d:], W_gate[2 * d:]], axis=1)
    gam = jnp.concatenate([gamma_mlpt, gamma_gate]).reshape(1, d2)
    bet = jnp.concatenate([beta_mlpt, beta_gate]).reshape(1, d2)
    src = edge_index[0]
    dst = edge_index[1]

    p_tab, d_tab = pl.pallas_call(
        functools.partial(_proj_body, d),
        out_shape=[jax.ShapeDtypeStruct((n_nodes, d), jnp.int32),
                   jax.ShapeDtypeStruct((n_nodes, d), jnp.int32)],
    )(x, w_src, w_dst)

    ew = n_edges // _NW
    blk = 80
    n_chunks = ew // blk
    nbuf = 3
    n_groups = (n_chunks + nbuf - 1) // nbuf
    mesh = plsc.VectorSubcoreMesh(core_axis_name="c", subcore_axis_name="s",
                                  num_cores=_NC, num_subcores=_NS)

    @functools.partial(
        pl.kernel,
        out_type=jax.ShapeDtypeStruct((n_edges, d), jnp.int32),
        mesh=mesh,
        scratch_types=[
            [pltpu.VMEM((blk,), jnp.int32)] * nbuf,
            [pltpu.VMEM((blk,), jnp.int32)] * nbuf,
            [pltpu.VMEM((blk, d), jnp.int32)] * nbuf,
            [pltpu.VMEM((blk, d), jnp.int32)] * nbuf,
            [pltpu.SemaphoreType.DMA] * nbuf,
            [pltpu.SemaphoreType.DMA] * nbuf,
            [pltpu.SemaphoreType.DMA] * nbuf,
        ],
    )
    def _gather_add(p_hbm, dn_hbm, src_hbm, dst_hbm, g_hbm,
                    sidx, didx, prow, drow, semp, semd, semw):
        wid = lax.axis_index("s") * _NC + lax.axis_index("c")
        base = wid * ew

        def start(i, b):
            off = base + i * blk
            pltpu.sync_copy(src_hbm.at[pl.ds(off, blk)], sidx[b])
            pltpu.sync_copy(dst_hbm.at[pl.ds(off, blk)], didx[b])
            pltpu.async_copy(p_hbm.at[sidx[b]], prow[b], semp[b])
            pltpu.async_copy(dn_hbm.at[didx[b]], drow[b], semd[b])

        def drain(i, b):
            off = base + i * blk
            pltpu.make_async_copy(p_hbm.at[sidx[b]], prow[b], semp[b]).wait()
            pltpu.make_async_copy(dn_hbm.at[didx[b]], drow[b], semd[b]).wait()
            pb = prow[b]
            db = drow[b]

            def row(r, carry):
                for j in range(d // 16):
                    sl = pl.ds(j * 16, 16)
                    bf = lambda w: lax.bitcast_convert_type(w, jnp.float32)
                    bi = lambda f: lax.bitcast_convert_type(f, jnp.int32)
                    pw = pb[r, sl]
                    dw = db[r, sl]
                    sm = bf(lax.shift_left(pw, 16)) + bf(lax.shift_left(dw, 16))
                    sg = bf(pw & _HI) + bf(dw & _HI)
                    pb[r, sl] = ((bi(sg) & _HI)
                                 | lax.shift_right_logical(bi(sm), 16))
                return carry

            lax.fori_loop(0, blk, row, 0)
            pltpu.async_copy(pb, g_hbm.at[pl.ds(off, blk)], semw[b])

        def wait_wb(i, b):
            off = base + i * blk
            pltpu.make_async_copy(
                prow[b], g_hbm.at[pl.ds(off, blk)], semw[b]).wait()

        for b in range(nbuf):
            if b < n_chunks:
                start(b, b)

        def group(gj, carry):
            for b in range(nbuf):
                i = gj * nbuf + b

                @pl.when(i < n_chunks)
                def _():
                    drain(i, b)
            for b in range(nbuf):
                i = gj * nbuf + b

                @pl.when(i + nbuf < n_chunks)
                def _():
                    wait_wb(i, b)
                    start(i + nbuf, b)
            return carry

        lax.fori_loop(0, n_groups, group, 0)
        for i in range(max(0, n_chunks - nbuf), n_chunks):
            wait_wb(i, i % nbuf)

    g_w = _gather_add(p_tab, d_tab, src, dst)

    be = 16000
    n_eblk = n_edges // be
    sums, sqs, y_w = pl.pallas_call(
        functools.partial(_stats_body, d),
        grid=(n_eblk,),
        in_specs=[
            pl.BlockSpec((be, d), lambda i: (i, 0)),
            pl.BlockSpec((be, d), lambda i: (i, 0)),
            pl.BlockSpec((d, d2), lambda i: (0, 0)),
        ],
        out_specs=[pl.BlockSpec((1, d2), lambda i: (0, 0)),
                   pl.BlockSpec((1, d2), lambda i: (0, 0)),
                   pl.BlockSpec((be, d), lambda i: (i, 0))],
        out_shape=[jax.ShapeDtypeStruct((1, d2), jnp.float32),
                   jax.ShapeDtypeStruct((1, d2), jnp.float32),
                   jax.ShapeDtypeStruct((n_edges, d), jnp.int32)],
    )(edge_attr, g_w, w_edge)

    m = pl.pallas_call(
        functools.partial(_act_body, float(n_edges), d),
        grid=(n_eblk,),
        in_specs=[
            pl.BlockSpec((be, d), lambda i: (i, 0)),
            pl.BlockSpec((1, d2), lambda i: (0, 0)),
            pl.BlockSpec((1, d2), lambda i: (0, 0)),
            pl.BlockSpec((1, d2), lambda i: (0, 0)),
            pl.BlockSpec((1, d2), lambda i: (0, 0)),
        ],
        out_specs=pl.BlockSpec((be, d), lambda i: (i, 0)),
        out_shape=jax.ShapeDtypeStruct((n_edges, d), jnp.float32),
    )(y_w, sums, sqs, gam, bet)

    n_pad = ((n_nodes + 8 * _NS - 1) // (8 * _NS)) * (8 * _NS)
    rows_per_tile = n_pad // _NS

    @functools.partial(
        pl.kernel,
        out_type=jax.ShapeDtypeStruct((_NC, n_pad, d), jnp.float32),
        mesh=mesh,
        scratch_types=[
            [pltpu.VMEM((blk,), jnp.int32)] * nbuf,
            [pltpu.VMEM((blk, d), jnp.float32)] * nbuf,
            pltpu.VMEM((64, d), jnp.float32),
            pltpu.VMEM_SHARED((n_pad, d), jnp.float32),
            [pltpu.SemaphoreType.DMA] * nbuf,
            [pltpu.SemaphoreType.DMA] * nbuf,
        ],
    )
    def _scatter_add(m_hbm, dst_hbm, out_hbm, didx, mrow, zbuf, agg_sh,
                     semm, semx):
        c = lax.axis_index("c")
        s = lax.axis_index("s")
        wid = s * _NC + c

        def zrow(r, carry):
            for j in range(d // 16):
                zbuf[r, pl.ds(j * 16, 16)] = jnp.zeros((16,), jnp.float32)
            return carry

        lax.fori_loop(0, 64, zrow, 0)
        n_zc = (rows_per_tile + 63) // 64

        def zcopy(i, carry):
            off = jnp.minimum(i * 64, rows_per_tile - 64)
            pltpu.sync_copy(zbuf,
                            agg_sh.at[pl.ds(s * rows_per_tile + off, 64)])
            return carry

        lax.fori_loop(0, n_zc, zcopy, 0)
        plsc.subcore_barrier()

        base = wid * ew

        def start(i, b):
            off = base + i * blk
            pltpu.async_copy(dst_hbm.at[pl.ds(off, blk)], didx[b], semx[b])
            pltpu.async_copy(m_hbm.at[pl.ds(off, blk)], mrow[b], semm[b])

        def drain(i, b):
            off = base + i * blk
            pltpu.make_async_copy(
                dst_hbm.at[pl.ds(off, blk)], didx[b], semx[b]).wait()
            pltpu.make_async_copy(
                m_hbm.at[pl.ds(off, blk)], mrow[b], semm[b]).wait()
            pltpu.sync_copy(mrow[b], agg_sh.at[didx[b]], add=True)

        for b in range(nbuf):
            if b < n_chunks:
                start(b, b)

        def group(gj, carry):
            for b in range(nbuf):
                i = gj * nbuf + b

                @pl.when(i < n_chunks)
                def _():
                    drain(i, b)

                    @pl.when(i + nbuf < n_chunks)
                    def _():
                        start(i + nbuf, b)
            return carry

        lax.fori_loop(0, n_groups, group, 0)
        plsc.subcore_barrier()
        pltpu.sync_copy(
            agg_sh.at[pl.ds(s * rows_per_tile, rows_per_tile)],
            out_hbm.at[c, pl.ds(s * rows_per_tile, rows_per_tile)])

    partials = _scatter_add(m, dst)

    node_out = pl.pallas_call(
        functools.partial(_final_body, n_nodes),
        out_shape=jax.ShapeDtypeStruct((n_nodes, d), jnp.float32),
    )(partials, x, gamma_node.reshape(1, d), beta_node.reshape(1, d))

    return (node_out, edge_attr)
